# baseline (speedup 1.0000x reference)
import jax
import jax.numpy as jnp
from jax.experimental import pallas as pl
from jax.experimental.pallas import tpu as pltpu

B, L, V, E, H = 16384, 20, 512, 64, 64
BLK = 1024
SUB = 512
NB = B // BLK


def _body(lens, ids_e, len_e, ids_d, len_d, emb_t, wie, whe, bie,
          wid, whd, bid, bhe, bhd, out, ge_ref, gd_ref, yt_ref):
    i = pl.program_id(0)
    me = lens[i, 0]
    md = lens[i, 1]

    @pl.when(i == 0)
    def _build_tables():
        embv = emb_t[:].astype(jnp.bfloat16)
        ge_ref[:] = (jax.lax.dot_general(
            wie[:].astype(jnp.bfloat16), embv, (((1,), (0,)), ((), ())),
            preferred_element_type=jnp.float32) + bie[:]).astype(jnp.bfloat16)
        gd_ref[:] = (jax.lax.dot_general(
            wid[:].astype(jnp.bfloat16), embv, (((1,), (0,)), ((), ())),
            preferred_element_type=jnp.float32) + bid[:]).astype(jnp.bfloat16)

    iota = jax.lax.broadcasted_iota(jnp.int32, (V, 1), 0).astype(jnp.int16)
    one_bf = jnp.ones((), jnp.bfloat16)
    zero_bf = jnp.zeros((), jnp.bfloat16)

    whe_ = whe[:].astype(jnp.bfloat16)
    whd_ = whd[:].astype(jnp.bfloat16)
    bheb = jnp.broadcast_to(bhe[:], (3 * H, SUB))
    bhdb = jnp.broadcast_to(bhd[:], (3 * H, SUB))

    lensA_e = len_e[0:1, 0:SUB]
    lensB_e = len_e[0:1, SUB:BLK]
    lensA_d = len_d[0:1, 0:SUB]
    lensB_d = len_d[0:1, SUB:BLK]

    def cell(ids_ref, g_ref, a0, t, wh, bhb, h):
        idc = ids_ref[pl.ds(t, 1), 0, a0:a0 + SUB]
        oh = jnp.where(idc == iota, one_bf, zero_bf)
        gi = jax.lax.dot_general(g_ref[:], oh, (((1,), (0,)), ((), ())),
                                 preferred_element_type=jnp.float32)
        gh = jax.lax.dot_general(wh, h.astype(jnp.bfloat16),
                                 (((1,), (0,)), ((), ())),
                                 preferred_element_type=jnp.float32) + bhb
        rz = jax.nn.sigmoid(gi[:2 * H, :] + gh[:2 * H, :])
        z = rz[H:, :]
        n = jnp.tanh(gi[2 * H:, :] + rz[:H, :] * gh[2 * H:, :])
        return (1.0 - z) * n + z * h

    def enc_body(t, hs):
        hA, hB = hs
        hnA = cell(ids_e, ge_ref, 0, t, whe_, bheb, hA)
        hnB = cell(ids_e, ge_ref, SUB, t, whe_, bheb, hB)
        hA = jnp.where(lensA_e > t, hnA, hA)
        hB = jnp.where(lensB_e > t, hnB, hB)
        return hA, hB

    def dec_body(t, hs):
        hA, hB = hs
        hnA = cell(ids_d, gd_ref, 0, t, whd_, bhdb, hA)
        hnB = cell(ids_d, gd_ref, SUB, t, whd_, bhdb, hB)
        mA = lensA_d > t
        mB = lensB_d > t
        yt_ref[pl.ds(t, 1), :, 0:SUB] = jnp.where(mA, hnA, 0.0).reshape(
            1, H, SUB)
        yt_ref[pl.ds(t, 1), :, SUB:BLK] = jnp.where(mB, hnB, 0.0).reshape(
            1, H, SUB)
        hA = jnp.where(mA, hnA, hA)
        hB = jnp.where(mB, hnB, hB)
        return hA, hB

    yt_ref[:] = jnp.zeros((L, H, BLK), jnp.float32)
    h0 = jnp.zeros((H, SUB), jnp.float32)
    hs = jax.lax.fori_loop(0, me, enc_body, (h0, h0))
    jax.lax.fori_loop(0, md, dec_body, hs)
    for t in range(L):
        out[:, t, :] = jnp.swapaxes(yt_ref[t], 0, 1)


def kernel(input, output, input_mask, output_mask, input_word_len,
           output_word_len, emb, w_ih_enc, w_hh_enc, b_ih_enc, b_hh_enc,
           w_ih_dec, w_hh_dec, b_ih_dec, b_hh_dec):
    in_len = input_word_len[:, 0]
    perm_in = jnp.argsort(-in_len)
    ids_e = jnp.take(input, perm_in, axis=0).astype(jnp.int16)
    len_e = jnp.take(in_len, perm_in).astype(jnp.int32)

    out_len = output_word_len[:, 0]
    perm_out = jnp.argsort(-out_len)
    ids_d = jnp.take(output, perm_out, axis=0).astype(jnp.int16)
    len_d = jnp.take(out_len, perm_out).astype(jnp.int32)

    ids_e3 = ids_e.T.reshape(L, 1, B)
    ids_d3 = ids_d.T.reshape(L, 1, B)

    maxlens = jnp.stack([len_e[::BLK], len_d[::BLK]], axis=1).astype(jnp.int32)

    full = lambda shape: pl.BlockSpec(shape, lambda i, *_: (0,) * len(shape))
    ids_spec = pl.BlockSpec((L, 1, BLK), lambda i, *_: (0, 0, i))
    len_spec = pl.BlockSpec((1, BLK), lambda i, *_: (0, i))

    grid_spec = pltpu.PrefetchScalarGridSpec(
        num_scalar_prefetch=1,
        grid=(NB,),
        in_specs=[
            ids_spec, len_spec, ids_spec, len_spec,
            full((E, V)),
            full((3 * H, E)), full((3 * H, H)), full((3 * H, 1)),
            full((3 * H, E)), full((3 * H, H)), full((3 * H, 1)),
            full((3 * H, 1)), full((3 * H, 1)),
        ],
        out_specs=pl.BlockSpec((BLK, L, H), lambda i, *_: (i, 0, 0)),
        scratch_shapes=[
            pltpu.VMEM((3 * H, V), jnp.bfloat16),
            pltpu.VMEM((3 * H, V), jnp.bfloat16),
            pltpu.VMEM((L, H, BLK), jnp.float32),
        ],
    )

    return pl.pallas_call(
        _body,
        grid_spec=grid_spec,
        out_shape=jax.ShapeDtypeStruct((B, L, H), jnp.float32),
    )(maxlens, ids_e3, len_e[None, :], ids_d3, len_d[None, :], emb.T,
      w_ih_enc, w_hh_enc, b_ih_enc.reshape(3 * H, 1),
      w_ih_dec, w_hh_dec, b_ih_dec.reshape(3 * H, 1),
      b_hh_enc.reshape(3 * H, 1), b_hh_dec.reshape(3 * H, 1))


# speedup vs baseline: 1.7164x; 1.7164x over previous
import jax
import jax.numpy as jnp
from jax.experimental import pallas as pl
from jax.experimental.pallas import tpu as pltpu

B, L, V, E, H = 16384, 20, 512, 64, 64
BLK = 2048
SUB = 1024
NB = B // BLK


def _body(lens, ids_e, len_e, ids_d, len_d, emb_t, wie, whe, bie,
          wid, whd, bid, bhe, bhd, out, ge_ref, gd_ref):
    i = pl.program_id(0)
    me = lens[i, 0]
    md = lens[i, 1]

    @pl.when(i == 0)
    def _build_tables():
        embv = emb_t[:].astype(jnp.bfloat16)
        ge_ref[:] = (jax.lax.dot_general(
            wie[:].astype(jnp.bfloat16), embv, (((1,), (0,)), ((), ())),
            preferred_element_type=jnp.float32) + bie[:]).astype(jnp.bfloat16)
        gd_ref[:] = (jax.lax.dot_general(
            wid[:].astype(jnp.bfloat16), embv, (((1,), (0,)), ((), ())),
            preferred_element_type=jnp.float32) + bid[:]).astype(jnp.bfloat16)

    iota = jax.lax.broadcasted_iota(jnp.int32, (V, 1), 0).astype(jnp.int16)
    one_bf = jnp.ones((), jnp.bfloat16)
    zero_bf = jnp.zeros((), jnp.bfloat16)

    whe_ = whe[:].astype(jnp.bfloat16)
    whd_ = whd[:].astype(jnp.bfloat16)
    bheb = jnp.broadcast_to(bhe[:], (3 * H, SUB))
    bhdb = jnp.broadcast_to(bhd[:], (3 * H, SUB))

    lensA_e = len_e[0:1, 0:SUB]
    lensB_e = len_e[0:1, SUB:BLK]
    lensA_d = len_d[0:1, 0:SUB]
    lensB_d = len_d[0:1, SUB:BLK]

    def cell(ids_ref, g_ref, a0, t, wh, bhb, h):
        idc = ids_ref[pl.ds(t, 1), 0, a0:a0 + SUB]
        oh = jnp.where(idc == iota, one_bf, zero_bf)
        gi = jax.lax.dot_general(g_ref[:], oh, (((1,), (0,)), ((), ())),
                                 preferred_element_type=jnp.float32)
        gh = jax.lax.dot_general(wh, h.astype(jnp.bfloat16),
                                 (((1,), (0,)), ((), ())),
                                 preferred_element_type=jnp.float32) + bhb
        rz = 0.5 * jnp.tanh(0.5 * (gi[:2 * H, :] + gh[:2 * H, :])) + 0.5
        z = rz[H:, :]
        n = jnp.tanh(gi[2 * H:, :] + rz[:H, :] * gh[2 * H:, :])
        return (1.0 - z) * n + z * h

    def enc_body(t, hs):
        hA, hB = hs
        hnA = cell(ids_e, ge_ref, 0, t, whe_, bheb, hA)
        hnB = cell(ids_e, ge_ref, SUB, t, whe_, bheb, hB)
        hA = jnp.where(lensA_e > t, hnA, hA)
        hB = jnp.where(lensB_e > t, hnB, hB)
        return hA, hB

    def dec_body(t, hs):
        hA, hB = hs
        hnA = cell(ids_d, gd_ref, 0, t, whd_, bhdb, hA)
        hnB = cell(ids_d, gd_ref, SUB, t, whd_, bhdb, hB)
        mA = lensA_d > t
        mB = lensB_d > t
        out[pl.ds(t, 1), :, 0:SUB] = jnp.where(mA, hnA, 0.0).reshape(
            1, H, SUB)
        out[pl.ds(t, 1), :, SUB:BLK] = jnp.where(mB, hnB, 0.0).reshape(
            1, H, SUB)
        hA = jnp.where(mA, hnA, hA)
        hB = jnp.where(mB, hnB, hB)
        return hA, hB

    out[:] = jnp.zeros((L, H, BLK), jnp.float32)
    h0 = jnp.zeros((H, SUB), jnp.float32)
    hs = jax.lax.fori_loop(0, me, enc_body, (h0, h0))
    jax.lax.fori_loop(0, md, dec_body, hs)


def kernel(input, output, input_mask, output_mask, input_word_len,
           output_word_len, emb, w_ih_enc, w_hh_enc, b_ih_enc, b_hh_enc,
           w_ih_dec, w_hh_dec, b_ih_dec, b_hh_dec):
    in_len = input_word_len[:, 0]
    perm_in = jnp.argsort(-in_len)
    ids_e = jnp.take(input, perm_in, axis=0).astype(jnp.int16)
    len_e = jnp.take(in_len, perm_in).astype(jnp.int32)

    out_len = output_word_len[:, 0]
    perm_out = jnp.argsort(-out_len)
    ids_d = jnp.take(output, perm_out, axis=0).astype(jnp.int16)
    len_d = jnp.take(out_len, perm_out).astype(jnp.int32)

    ids_e3 = ids_e.T.reshape(L, 1, B)
    ids_d3 = ids_d.T.reshape(L, 1, B)

    maxlens = jnp.stack([len_e[::BLK], len_d[::BLK]], axis=1).astype(jnp.int32)

    full = lambda shape: pl.BlockSpec(shape, lambda i, *_: (0,) * len(shape))
    ids_spec = pl.BlockSpec((L, 1, BLK), lambda i, *_: (0, 0, i))
    len_spec = pl.BlockSpec((1, BLK), lambda i, *_: (0, i))

    grid_spec = pltpu.PrefetchScalarGridSpec(
        num_scalar_prefetch=1,
        grid=(NB,),
        in_specs=[
            ids_spec, len_spec, ids_spec, len_spec,
            full((E, V)),
            full((3 * H, E)), full((3 * H, H)), full((3 * H, 1)),
            full((3 * H, E)), full((3 * H, H)), full((3 * H, 1)),
            full((3 * H, 1)), full((3 * H, 1)),
        ],
        out_specs=pl.BlockSpec((L, H, BLK), lambda i, *_: (0, 0, i)),
        scratch_shapes=[
            pltpu.VMEM((3 * H, V), jnp.bfloat16),
            pltpu.VMEM((3 * H, V), jnp.bfloat16),
        ],
    )

    out_t = pl.pallas_call(
        _body,
        grid_spec=grid_spec,
        out_shape=jax.ShapeDtypeStruct((L, H, B), jnp.float32),
    )(maxlens, ids_e3, len_e[None, :], ids_d3, len_d[None, :], emb.T,
      w_ih_enc, w_hh_enc, b_ih_enc.reshape(3 * H, 1),
      w_ih_dec, w_hh_dec, b_ih_dec.reshape(3 * H, 1),
      b_hh_enc.reshape(3 * H, 1), b_hh_dec.reshape(3 * H, 1))
    return jnp.transpose(out_t, (2, 0, 1))


# speedup vs baseline: 1.7841x; 1.0394x over previous
import jax
import jax.numpy as jnp
from jax.experimental import pallas as pl
from jax.experimental.pallas import tpu as pltpu

B, L, V, E, H = 16384, 20, 512, 64, 64
BLK = 2048
SUB = 512
NB = B // BLK


def _body(lens, ids_e, len_e, ids_d, len_d, emb_t, wie, whe, bie,
          wid, whd, bid, bhe, bhd, out, ge_ref, gd_ref):
    i = pl.program_id(0)
    me = lens[i, 0]
    md = lens[i, 1]

    @pl.when(i == 0)
    def _build_tables():
        embv = emb_t[:].astype(jnp.bfloat16)
        ge_ref[:] = (jax.lax.dot_general(
            wie[:].astype(jnp.bfloat16), embv, (((1,), (0,)), ((), ())),
            preferred_element_type=jnp.float32) + bie[:]).astype(jnp.bfloat16)
        gd_ref[:] = (jax.lax.dot_general(
            wid[:].astype(jnp.bfloat16), embv, (((1,), (0,)), ((), ())),
            preferred_element_type=jnp.float32) + bid[:]).astype(jnp.bfloat16)

    iota = jax.lax.broadcasted_iota(jnp.int32, (V, 1), 0).astype(jnp.int16)
    one_bf = jnp.ones((), jnp.bfloat16)
    zero_bf = jnp.zeros((), jnp.bfloat16)

    whe_ = whe[:].astype(jnp.bfloat16)
    whd_ = whd[:].astype(jnp.bfloat16)
    bheb = jnp.broadcast_to(bhe[:], (3 * H, SUB))
    bhdb = jnp.broadcast_to(bhd[:], (3 * H, SUB))

    offs = tuple(range(0, BLK, SUB))
    lens_e = tuple(len_e[0:1, a:a + SUB] for a in offs)
    lens_d = tuple(len_d[0:1, a:a + SUB] for a in offs)

    def cell(ids_ref, g_ref, a0, t, wh, bhb, h):
        idc = ids_ref[pl.ds(t, 1), 0, a0:a0 + SUB]
        oh = jnp.where(idc == iota, one_bf, zero_bf)
        gi = jax.lax.dot_general(g_ref[:], oh, (((1,), (0,)), ((), ())),
                                 preferred_element_type=jnp.float32)
        gh = jax.lax.dot_general(wh, h.astype(jnp.bfloat16),
                                 (((1,), (0,)), ((), ())),
                                 preferred_element_type=jnp.float32) + bhb
        rz = 0.5 * jnp.tanh(0.5 * (gi[:2 * H, :] + gh[:2 * H, :])) + 0.5
        z = rz[H:, :]
        n = jnp.tanh(gi[2 * H:, :] + rz[:H, :] * gh[2 * H:, :])
        return (1.0 - z) * n + z * h

    def enc_body(t, hs):
        hn = [cell(ids_e, ge_ref, a, t, whe_, bheb, h)
              for a, h in zip(offs, hs)]
        return tuple(jnp.where(lv > t, hnew, h)
                     for lv, hnew, h in zip(lens_e, hn, hs))

    def dec_body(t, hs):
        hn = [cell(ids_d, gd_ref, a, t, whd_, bhdb, h)
              for a, h in zip(offs, hs)]
        ms = [lv > t for lv in lens_d]
        for a, m, hnew in zip(offs, ms, hn):
            out[pl.ds(t, 1), :, a:a + SUB] = jnp.where(
                m, hnew, 0.0).reshape(1, H, SUB)
        return tuple(jnp.where(m, hnew, h)
                     for m, hnew, h in zip(ms, hn, hs))

    out[:] = jnp.zeros((L, H, BLK), jnp.float32)
    h0 = tuple(jnp.zeros((H, SUB), jnp.float32) for _ in offs)
    hs = jax.lax.fori_loop(0, me, enc_body, h0)
    jax.lax.fori_loop(0, md, dec_body, hs)


def kernel(input, output, input_mask, output_mask, input_word_len,
           output_word_len, emb, w_ih_enc, w_hh_enc, b_ih_enc, b_hh_enc,
           w_ih_dec, w_hh_dec, b_ih_dec, b_hh_dec):
    in_len = input_word_len[:, 0]
    perm_in = jnp.argsort(-in_len)
    ids_e = jnp.take(input, perm_in, axis=0).astype(jnp.int16)
    len_e = jnp.take(in_len, perm_in).astype(jnp.int32)

    out_len = output_word_len[:, 0]
    perm_out = jnp.argsort(-out_len)
    ids_d = jnp.take(output, perm_out, axis=0).astype(jnp.int16)
    len_d = jnp.take(out_len, perm_out).astype(jnp.int32)

    ids_e3 = ids_e.T.reshape(L, 1, B)
    ids_d3 = ids_d.T.reshape(L, 1, B)

    maxlens = jnp.stack([len_e[::BLK], len_d[::BLK]], axis=1).astype(jnp.int32)

    full = lambda shape: pl.BlockSpec(shape, lambda i, *_: (0,) * len(shape))
    ids_spec = pl.BlockSpec((L, 1, BLK), lambda i, *_: (0, 0, i))
    len_spec = pl.BlockSpec((1, BLK), lambda i, *_: (0, i))

    grid_spec = pltpu.PrefetchScalarGridSpec(
        num_scalar_prefetch=1,
        grid=(NB,),
        in_specs=[
            ids_spec, len_spec, ids_spec, len_spec,
            full((E, V)),
            full((3 * H, E)), full((3 * H, H)), full((3 * H, 1)),
            full((3 * H, E)), full((3 * H, H)), full((3 * H, 1)),
            full((3 * H, 1)), full((3 * H, 1)),
        ],
        out_specs=pl.BlockSpec((L, H, BLK), lambda i, *_: (0, 0, i)),
        scratch_shapes=[
            pltpu.VMEM((3 * H, V), jnp.bfloat16),
            pltpu.VMEM((3 * H, V), jnp.bfloat16),
        ],
    )

    out_t = pl.pallas_call(
        _body,
        grid_spec=grid_spec,
        out_shape=jax.ShapeDtypeStruct((L, H, B), jnp.float32),
    )(maxlens, ids_e3, len_e[None, :], ids_d3, len_d[None, :], emb.T,
      w_ih_enc, w_hh_enc, b_ih_enc.reshape(3 * H, 1),
      w_ih_dec, w_hh_dec, b_ih_dec.reshape(3 * H, 1),
      b_hh_enc.reshape(3 * H, 1), b_hh_dec.reshape(3 * H, 1))
    return jnp.transpose(out_t, (2, 0, 1))
